# initial kernel scaffold (unmeasured)
import functools

import jax
import jax.numpy as jnp
from jax import lax
from jax.experimental import pallas as pl
from jax.experimental.pallas import tpu as pltpu

N_DEV = 4
NT = 4


def kernel(x, w_mat, scale_x, scale_w):
    m_total, k_shard = x.shape
    k2, n_total = w_mat.shape
    assert k2 == k_shard
    m_chunk = m_total // N_DEV
    nt = n_total // NT

    def body(x_ref, w_ref, sx_ref, sw_ref, out_ref,
             comm_ref, send_sems, recv_sems, credit_sem):
        my = lax.axis_index("i")
        left = (my + N_DEV - 1) % N_DEV
        right = (my + 1) % N_DEV

        barrier_sem = pltpu.get_barrier_semaphore()
        for nbr in (left, right):
            pl.semaphore_signal(
                barrier_sem, inc=1,
                device_id=(nbr,), device_id_type=pl.DeviceIdType.MESH,
            )
        pl.semaphore_wait(barrier_sem, 2)

        scale = sx_ref[0] * sw_ref[0]

        def partial(c, t):
            xs = x_ref[pl.ds(c * m_chunk, m_chunk), :]
            ws = w_ref[:, pl.ds(t * nt, nt)]
            return lax.dot_general(
                xs, ws, (((1,), (0,)), ((), ())),
                preferred_element_type=jnp.int32,
            )

        for t in range(NT):
            if t > 0:
                pl.semaphore_wait(credit_sem, 1)

            comm_ref[0, :, :] = partial((my + N_DEV - 1) % N_DEV, t)

            for h in range(N_DEV - 1):
                rdma = pltpu.make_async_remote_copy(
                    src_ref=comm_ref.at[h],
                    dst_ref=comm_ref.at[h + 1],
                    send_sem=send_sems.at[h],
                    recv_sem=recv_sems.at[h],
                    device_id=(right,),
                    device_id_type=pl.DeviceIdType.MESH,
                )
                rdma.start()
                rdma.wait()

                c = (my + 2 * N_DEV - 2 - h) % N_DEV
                if h < N_DEV - 2:
                    comm_ref[h + 1, :, :] += partial(c, t)
                else:
                    acc = comm_ref[h + 1, :, :] + partial(c, t)
                    out_ref[:, pl.ds(t * nt, nt)] = (
                        acc.astype(jnp.float32) * scale
                    )

            if t < NT - 1:
                pl.semaphore_signal(
                    credit_sem, inc=1,
                    device_id=(left,), device_id_type=pl.DeviceIdType.MESH,
                )

        @functools.partial(
            pl.run_scoped, second_barrier=pltpu.SemaphoreType.REGULAR
        )
        def _(second_barrier):
            for nbr in (left, right):
                pl.semaphore_signal(
                    second_barrier, inc=1,
                    device_id=(nbr,), device_id_type=pl.DeviceIdType.MESH,
                )
            pl.semaphore_wait(second_barrier, 2)

    return pl.pallas_call(
        body,
        out_shape=jax.ShapeDtypeStruct((m_chunk, n_total), jnp.float32),
        in_specs=[
            pl.BlockSpec(memory_space=pltpu.VMEM),
            pl.BlockSpec(memory_space=pltpu.VMEM),
            pl.BlockSpec(memory_space=pltpu.SMEM),
            pl.BlockSpec(memory_space=pltpu.SMEM),
        ],
        out_specs=pl.BlockSpec(memory_space=pltpu.VMEM),
        scratch_shapes=[
            pltpu.VMEM((N_DEV, m_chunk, nt), jnp.int32),
            pltpu.SemaphoreType.DMA((N_DEV - 1,)),
            pltpu.SemaphoreType.DMA((N_DEV - 1,)),
            pltpu.SemaphoreType.REGULAR,
        ],
        compiler_params=pltpu.CompilerParams(collective_id=0),
    )(x, w_mat, scale_x, scale_w)


# baseline (device time: 1254437 ns/iter reference)
import functools

import jax
import jax.numpy as jnp
from jax import lax
from jax.experimental import pallas as pl
from jax.experimental.pallas import tpu as pltpu

N_DEV = 4
NT = 8


def kernel(x, w_mat, scale_x, scale_w):
    m_total, k_shard = x.shape
    k2, n_total = w_mat.shape
    assert k2 == k_shard
    m_chunk = m_total // N_DEV
    nt = n_total // NT

    def body(x_ref, w_ref, sx_ref, sw_ref, out_hbm,
             comm_ref, stage_ref, send_sems, recv_sems, copy_sems,
             credit_sem):
        my = lax.axis_index("i")
        left = (my + N_DEV - 1) % N_DEV
        right = (my + 1) % N_DEV

        barrier_sem = pltpu.get_barrier_semaphore()
        for nbr in (left, right):
            pl.semaphore_signal(
                barrier_sem, inc=1,
                device_id=(nbr,), device_id_type=pl.DeviceIdType.MESH,
            )
        pl.semaphore_wait(barrier_sem, 2)

        scale = sx_ref[0] * sw_ref[0]

        def partial(c, t):
            xs = x_ref[pl.ds(c * m_chunk, m_chunk), :]
            ws = w_ref[:, pl.ds(t * nt, nt)]
            return lax.dot_general(
                xs, ws, (((1,), (0,)), ((), ())),
                preferred_element_type=jnp.int32,
            )

        for t in range(NT):
            if t > 0:
                pl.semaphore_wait(credit_sem, 1)

            comm_ref[0, :, :] = partial((my + N_DEV - 1) % N_DEV, t)

            for h in range(N_DEV - 1):
                rdma = pltpu.make_async_remote_copy(
                    src_ref=comm_ref.at[h],
                    dst_ref=comm_ref.at[h + 1],
                    send_sem=send_sems.at[h],
                    recv_sem=recv_sems.at[h],
                    device_id=(right,),
                    device_id_type=pl.DeviceIdType.MESH,
                )
                rdma.start()
                rdma.wait()

                c = (my + 2 * N_DEV - 2 - h) % N_DEV
                if h < N_DEV - 2:
                    comm_ref[h + 1, :, :] += partial(c, t)
                else:
                    acc = comm_ref[h + 1, :, :] + partial(c, t)
                    slot = t % 2
                    stage_ref[slot, :, :] = acc.astype(jnp.float32) * scale
                    copy = pltpu.make_async_copy(
                        stage_ref.at[slot],
                        out_hbm.at[:, pl.ds(t * nt, nt)],
                        copy_sems.at[slot],
                    )
                    copy.start()
                    copy.wait()

            if t < NT - 1:
                pl.semaphore_signal(
                    credit_sem, inc=1,
                    device_id=(left,), device_id_type=pl.DeviceIdType.MESH,
                )

        @functools.partial(
            pl.run_scoped, second_barrier=pltpu.SemaphoreType.REGULAR
        )
        def _(second_barrier):
            for nbr in (left, right):
                pl.semaphore_signal(
                    second_barrier, inc=1,
                    device_id=(nbr,), device_id_type=pl.DeviceIdType.MESH,
                )
            pl.semaphore_wait(second_barrier, 2)

    return pl.pallas_call(
        body,
        out_shape=jax.ShapeDtypeStruct((m_chunk, n_total), jnp.float32),
        in_specs=[
            pl.BlockSpec(memory_space=pltpu.VMEM),
            pl.BlockSpec(memory_space=pltpu.VMEM),
            pl.BlockSpec(memory_space=pltpu.SMEM),
            pl.BlockSpec(memory_space=pltpu.SMEM),
        ],
        out_specs=pl.BlockSpec(memory_space=pl.ANY),
        scratch_shapes=[
            pltpu.VMEM((N_DEV, m_chunk, nt), jnp.int32),
            pltpu.VMEM((2, m_chunk, nt), jnp.float32),
            pltpu.SemaphoreType.DMA((N_DEV - 1,)),
            pltpu.SemaphoreType.DMA((N_DEV - 1,)),
            pltpu.SemaphoreType.DMA((2,)),
            pltpu.SemaphoreType.REGULAR,
        ],
        compiler_params=pltpu.CompilerParams(collective_id=0),
    )(x, w_mat, scale_x, scale_w)


# device time: 626783 ns/iter; 2.0014x vs baseline; 2.0014x over previous
import functools

import jax
import jax.numpy as jnp
from jax import lax
from jax.experimental import pallas as pl
from jax.experimental.pallas import tpu as pltpu

N_DEV = 4
NT = 8
N_ROUNDS = NT // 2


def kernel(x, w_mat, scale_x, scale_w):
    m_total, k_shard = x.shape
    k2, n_total = w_mat.shape
    assert k2 == k_shard
    m_chunk = m_total // N_DEV
    nt = n_total // NT

    def body(x_ref, w_ref, sx_ref, sw_ref, out_hbm,
             comm_r, comm_l, stage_ref,
             send_sems_r, recv_sems_r, send_sems_l, recv_sems_l,
             copy_sems, credit_r, credit_l):
        my = lax.axis_index("i")
        left = (my + N_DEV - 1) % N_DEV
        right = (my + 1) % N_DEV

        barrier_sem = pltpu.get_barrier_semaphore()
        for nbr in (left, right):
            pl.semaphore_signal(
                barrier_sem, inc=1,
                device_id=(nbr,), device_id_type=pl.DeviceIdType.MESH,
            )
        pl.semaphore_wait(barrier_sem, 2)

        scale = sx_ref[0] * sw_ref[0]

        def partial(c, t):
            xs = x_ref[pl.ds(c * m_chunk, m_chunk), :]
            ws = w_ref[:, pl.ds(t * nt, nt)]
            return lax.dot_general(
                xs, ws, (((1,), (0,)), ((), ())),
                preferred_element_type=jnp.int32,
            )

        pending_copy = {}

        def store_tile(acc, t, slot):
            if slot in pending_copy:
                pending_copy[slot].wait()
            stage_ref[slot, :, :] = acc.astype(jnp.float32) * scale
            copy = pltpu.make_async_copy(
                stage_ref.at[slot],
                out_hbm.at[:, pl.ds(t * nt, nt)],
                copy_sems.at[slot],
            )
            copy.start()
            pending_copy[slot] = copy

        for r in range(N_ROUNDS):
            t_r = r
            t_l = N_ROUNDS + r

            if r > 0:
                pl.semaphore_wait(credit_r, 1)
                pl.semaphore_wait(credit_l, 1)

            comm_r[0, :, :] = partial((my + N_DEV - 1) % N_DEV, t_r)
            comm_l[0, :, :] = partial((my + 1) % N_DEV, t_l)

            for h in range(N_DEV - 1):
                rdma_r = pltpu.make_async_remote_copy(
                    src_ref=comm_r.at[h],
                    dst_ref=comm_r.at[h + 1],
                    send_sem=send_sems_r.at[h],
                    recv_sem=recv_sems_r.at[h],
                    device_id=(right,),
                    device_id_type=pl.DeviceIdType.MESH,
                )
                rdma_l = pltpu.make_async_remote_copy(
                    src_ref=comm_l.at[h],
                    dst_ref=comm_l.at[h + 1],
                    send_sem=send_sems_l.at[h],
                    recv_sem=recv_sems_l.at[h],
                    device_id=(left,),
                    device_id_type=pl.DeviceIdType.MESH,
                )
                rdma_r.start()
                rdma_l.start()

                c_r = (my + 2 * N_DEV - 2 - h) % N_DEV
                c_l = (my + 2 + h) % N_DEV
                p_r = partial(c_r, t_r)
                p_l = partial(c_l, t_l)

                rdma_r.wait()
                if h < N_DEV - 2:
                    comm_r[h + 1, :, :] += p_r
                else:
                    store_tile(comm_r[h + 1, :, :] + p_r, t_r, 0)

                rdma_l.wait()
                if h < N_DEV - 2:
                    comm_l[h + 1, :, :] += p_l
                else:
                    store_tile(comm_l[h + 1, :, :] + p_l, t_l, 1)

            if r < N_ROUNDS - 1:
                pl.semaphore_signal(
                    credit_r, inc=1,
                    device_id=(left,), device_id_type=pl.DeviceIdType.MESH,
                )
                pl.semaphore_signal(
                    credit_l, inc=1,
                    device_id=(right,), device_id_type=pl.DeviceIdType.MESH,
                )

        for copy in pending_copy.values():
            copy.wait()

        @functools.partial(
            pl.run_scoped, second_barrier=pltpu.SemaphoreType.REGULAR
        )
        def _(second_barrier):
            for nbr in (left, right):
                pl.semaphore_signal(
                    second_barrier, inc=1,
                    device_id=(nbr,), device_id_type=pl.DeviceIdType.MESH,
                )
            pl.semaphore_wait(second_barrier, 2)

    return pl.pallas_call(
        body,
        out_shape=jax.ShapeDtypeStruct((m_chunk, n_total), jnp.float32),
        in_specs=[
            pl.BlockSpec(memory_space=pltpu.VMEM),
            pl.BlockSpec(memory_space=pltpu.VMEM),
            pl.BlockSpec(memory_space=pltpu.SMEM),
            pl.BlockSpec(memory_space=pltpu.SMEM),
        ],
        out_specs=pl.BlockSpec(memory_space=pl.ANY),
        scratch_shapes=[
            pltpu.VMEM((N_DEV, m_chunk, nt), jnp.int32),
            pltpu.VMEM((N_DEV, m_chunk, nt), jnp.int32),
            pltpu.VMEM((2, m_chunk, nt), jnp.float32),
            pltpu.SemaphoreType.DMA((N_DEV - 1,)),
            pltpu.SemaphoreType.DMA((N_DEV - 1,)),
            pltpu.SemaphoreType.DMA((N_DEV - 1,)),
            pltpu.SemaphoreType.DMA((N_DEV - 1,)),
            pltpu.SemaphoreType.DMA((2,)),
            pltpu.SemaphoreType.REGULAR,
            pltpu.SemaphoreType.REGULAR,
        ],
        compiler_params=pltpu.CompilerParams(
            collective_id=0,
            vmem_limit_bytes=63 * 1024 * 1024,
        ),
    )(x, w_mat, scale_x, scale_w)


# device time: 603997 ns/iter; 2.0769x vs baseline; 1.0377x over previous
import functools

import jax
import jax.numpy as jnp
from jax import lax
from jax.experimental import pallas as pl
from jax.experimental.pallas import tpu as pltpu

N_DEV = 4
NT = 8
N_ROUNDS = NT // 2
S = 2


def kernel(x, w_mat, scale_x, scale_w):
    m_total, k_shard = x.shape
    k2, n_total = w_mat.shape
    assert k2 == k_shard
    m_chunk = m_total // N_DEV
    nt = n_total // NT
    ms = m_chunk // S

    def body(x_ref, w_ref, sx_ref, sw_ref, out_hbm,
             comm_r, comm_l, stage_ref,
             send_sems_r, recv_sems_r, send_sems_l, recv_sems_l,
             copy_sems, credit_r, credit_l):
        my = lax.axis_index("i")
        left = (my + N_DEV - 1) % N_DEV
        right = (my + 1) % N_DEV

        barrier_sem = pltpu.get_barrier_semaphore()
        for nbr in (left, right):
            pl.semaphore_signal(
                barrier_sem, inc=1,
                device_id=(nbr,), device_id_type=pl.DeviceIdType.MESH,
            )
        pl.semaphore_wait(barrier_sem, 2)

        scale = sx_ref[0] * sw_ref[0]

        def partial(c, t):
            xs = x_ref[pl.ds(c * m_chunk, m_chunk), :]
            ws = w_ref[:, pl.ds(t * nt, nt)]
            return lax.dot_general(
                xs, ws, (((1,), (0,)), ((), ())),
                preferred_element_type=jnp.int32,
            )

        dirs = {
            "r": (comm_r, send_sems_r, recv_sems_r),
            "l": (comm_l, send_sems_l, recv_sems_l),
        }
        pending_send = {}
        pending_copy = {}

        def start_send(tag, h, s, dev):
            comm, ssems, rsems = dirs[tag]
            rdma = pltpu.make_async_remote_copy(
                src_ref=comm.at[h, pl.ds(s * ms, ms), :],
                dst_ref=comm.at[h + 1, pl.ds(s * ms, ms), :],
                send_sem=ssems.at[h, s],
                recv_sem=rsems.at[h, s],
                device_id=(dev,),
                device_id_type=pl.DeviceIdType.MESH,
            )
            key = (tag, h, s)
            if key in pending_send:
                pending_send[key].wait_send()
            rdma.start()
            pending_send[key] = rdma
            return rdma

        for r in range(N_ROUNDS):
            t_r = r
            t_l = N_ROUNDS + r

            if r > 0:
                pl.semaphore_wait(credit_r, 1)
                pl.semaphore_wait(credit_l, 1)

            comm_r[0, :, :] = partial((my + N_DEV - 1) % N_DEV, t_r)
            comm_l[0, :, :] = partial((my + 1) % N_DEV, t_l)

            cur = {}
            for s in range(S):
                cur[("r", 0, s)] = start_send("r", 0, s, right)
                cur[("l", 0, s)] = start_send("l", 0, s, left)

            for h in range(N_DEV - 1):
                last = h == N_DEV - 2
                p_r = partial((my + 2 * N_DEV - 2 - h) % N_DEV, t_r)
                p_l = partial((my + 2 + h) % N_DEV, t_l)

                for s in range(S):
                    rs = pl.ds(s * ms, ms)
                    pv = slice(s * ms, (s + 1) * ms)

                    cur[("r", h, s)].wait_recv()
                    if not last:
                        comm_r[h + 1, rs, :] = comm_r[h + 1, rs, :] + p_r[pv, :]
                        cur[("r", h + 1, s)] = start_send("r", h + 1, s, right)
                    else:
                        if s == 0 and 0 in pending_copy:
                            pending_copy[0].wait()
                        stage_ref[0, rs, :] = (
                            (comm_r[h + 1, rs, :] + p_r[pv, :])
                            .astype(jnp.float32) * scale
                        )

                    cur[("l", h, s)].wait_recv()
                    if not last:
                        comm_l[h + 1, rs, :] = comm_l[h + 1, rs, :] + p_l[pv, :]
                        cur[("l", h + 1, s)] = start_send("l", h + 1, s, left)
                    else:
                        if s == 0 and 1 in pending_copy:
                            pending_copy[1].wait()
                        stage_ref[1, rs, :] = (
                            (comm_l[h + 1, rs, :] + p_l[pv, :])
                            .astype(jnp.float32) * scale
                        )

            for slot, t in ((0, t_r), (1, t_l)):
                copy = pltpu.make_async_copy(
                    stage_ref.at[slot],
                    out_hbm.at[:, pl.ds(t * nt, nt)],
                    copy_sems.at[slot],
                )
                copy.start()
                pending_copy[slot] = copy

            if r < N_ROUNDS - 1:
                pl.semaphore_signal(
                    credit_r, inc=1,
                    device_id=(left,), device_id_type=pl.DeviceIdType.MESH,
                )
                pl.semaphore_signal(
                    credit_l, inc=1,
                    device_id=(right,), device_id_type=pl.DeviceIdType.MESH,
                )

        for rdma in pending_send.values():
            rdma.wait_send()
        for copy in pending_copy.values():
            copy.wait()

        @functools.partial(
            pl.run_scoped, second_barrier=pltpu.SemaphoreType.REGULAR
        )
        def _(second_barrier):
            for nbr in (left, right):
                pl.semaphore_signal(
                    second_barrier, inc=1,
                    device_id=(nbr,), device_id_type=pl.DeviceIdType.MESH,
                )
            pl.semaphore_wait(second_barrier, 2)

    return pl.pallas_call(
        body,
        out_shape=jax.ShapeDtypeStruct((m_chunk, n_total), jnp.float32),
        in_specs=[
            pl.BlockSpec(memory_space=pltpu.VMEM),
            pl.BlockSpec(memory_space=pltpu.VMEM),
            pl.BlockSpec(memory_space=pltpu.SMEM),
            pl.BlockSpec(memory_space=pltpu.SMEM),
        ],
        out_specs=pl.BlockSpec(memory_space=pl.ANY),
        scratch_shapes=[
            pltpu.VMEM((N_DEV, m_chunk, nt), jnp.int32),
            pltpu.VMEM((N_DEV, m_chunk, nt), jnp.int32),
            pltpu.VMEM((2, m_chunk, nt), jnp.float32),
            pltpu.SemaphoreType.DMA((N_DEV - 1, S)),
            pltpu.SemaphoreType.DMA((N_DEV - 1, S)),
            pltpu.SemaphoreType.DMA((N_DEV - 1, S)),
            pltpu.SemaphoreType.DMA((N_DEV - 1, S)),
            pltpu.SemaphoreType.DMA((2,)),
            pltpu.SemaphoreType.REGULAR,
            pltpu.SemaphoreType.REGULAR,
        ],
        compiler_params=pltpu.CompilerParams(
            collective_id=0,
            vmem_limit_bytes=63 * 1024 * 1024,
        ),
    )(x, w_mat, scale_x, scale_w)


# device time: 602225 ns/iter; 2.0830x vs baseline; 1.0029x over previous
import functools

import jax
import jax.numpy as jnp
from jax import lax
from jax.experimental import pallas as pl
from jax.experimental.pallas import tpu as pltpu

N_DEV = 4
NT = 8
N_ROUNDS = NT // 2
S = 2


def kernel(x, w_mat, scale_x, scale_w):
    m_total, k_shard = x.shape
    k2, n_total = w_mat.shape
    assert k2 == k_shard
    m_chunk = m_total // N_DEV
    nt = n_total // NT
    ms = m_chunk // S

    def body(x_ref, w_ref, sx_ref, sw_ref, out_hbm,
             comm_r, comm_l, stage_ref,
             send_sems_r, recv_sems_r, send_sems_l, recv_sems_l,
             copy_sems, credit_r, credit_l):
        my = lax.axis_index("i")
        left = (my + N_DEV - 1) % N_DEV
        right = (my + 1) % N_DEV

        barrier_sem = pltpu.get_barrier_semaphore()
        for nbr in (left, right):
            pl.semaphore_signal(
                barrier_sem, inc=1,
                device_id=(nbr,), device_id_type=pl.DeviceIdType.MESH,
            )
        pl.semaphore_wait(barrier_sem, 2)

        scale = sx_ref[0] * sw_ref[0]

        def partial(c, t):
            xs = x_ref[pl.ds(c * m_chunk, m_chunk), :]
            ws = w_ref[:, pl.ds(t * nt, nt)]
            return lax.dot_general(
                xs, ws, (((1,), (0,)), ((), ())),
                preferred_element_type=jnp.int32,
            )

        dirs = {
            "r": (comm_r, send_sems_r, recv_sems_r),
            "l": (comm_l, send_sems_l, recv_sems_l),
        }
        pending_send = {}
        pending_copy = {}

        def start_send(tag, h, s, dev):
            comm, ssems, rsems = dirs[tag]
            rdma = pltpu.make_async_remote_copy(
                src_ref=comm.at[h, pl.ds(s * ms, ms), :],
                dst_ref=comm.at[h + 1, pl.ds(s * ms, ms), :],
                send_sem=ssems.at[h, s],
                recv_sem=rsems.at[h, s],
                device_id=(dev,),
                device_id_type=pl.DeviceIdType.MESH,
            )
            key = (tag, h, s)
            if key in pending_send:
                pending_send[key].wait_send()
            rdma.start()
            pending_send[key] = rdma
            return rdma

        for r in range(N_ROUNDS):
            t_r = r
            t_l = N_ROUNDS + r

            comm_r[0, :, :] = partial((my + N_DEV - 1) % N_DEV, t_r)
            comm_l[0, :, :] = partial((my + 1) % N_DEV, t_l)

            if r > 0:
                pl.semaphore_wait(credit_r, 1)
                pl.semaphore_wait(credit_l, 1)

            cur = {}
            for s in range(S):
                cur[("r", 0, s)] = start_send("r", 0, s, right)
                cur[("l", 0, s)] = start_send("l", 0, s, left)

            for h in range(N_DEV - 1):
                last = h == N_DEV - 2
                p_r = partial((my + 2 * N_DEV - 2 - h) % N_DEV, t_r)
                p_l = partial((my + 2 + h) % N_DEV, t_l)

                for s in range(S):
                    rs = pl.ds(s * ms, ms)
                    pv = slice(s * ms, (s + 1) * ms)

                    cur[("r", h, s)].wait_recv()
                    if not last:
                        comm_r[h + 1, rs, :] = comm_r[h + 1, rs, :] + p_r[pv, :]
                        cur[("r", h + 1, s)] = start_send("r", h + 1, s, right)
                    else:
                        if s == 0 and 0 in pending_copy:
                            pending_copy[0].wait()
                        stage_ref[0, rs, :] = (
                            (comm_r[h + 1, rs, :] + p_r[pv, :])
                            .astype(jnp.float32) * scale
                        )

                    cur[("l", h, s)].wait_recv()
                    if not last:
                        comm_l[h + 1, rs, :] = comm_l[h + 1, rs, :] + p_l[pv, :]
                        cur[("l", h + 1, s)] = start_send("l", h + 1, s, left)
                    else:
                        if s == 0 and 1 in pending_copy:
                            pending_copy[1].wait()
                        stage_ref[1, rs, :] = (
                            (comm_l[h + 1, rs, :] + p_l[pv, :])
                            .astype(jnp.float32) * scale
                        )

            for key in list(pending_send):
                pending_send.pop(key).wait_send()
            if r < N_ROUNDS - 1:
                pl.semaphore_signal(
                    credit_r, inc=1,
                    device_id=(left,), device_id_type=pl.DeviceIdType.MESH,
                )
                pl.semaphore_signal(
                    credit_l, inc=1,
                    device_id=(right,), device_id_type=pl.DeviceIdType.MESH,
                )

            for slot, t in ((0, t_r), (1, t_l)):
                copy = pltpu.make_async_copy(
                    stage_ref.at[slot],
                    out_hbm.at[:, pl.ds(t * nt, nt)],
                    copy_sems.at[slot],
                )
                copy.start()
                pending_copy[slot] = copy

        for copy in pending_copy.values():
            copy.wait()

        @functools.partial(
            pl.run_scoped, second_barrier=pltpu.SemaphoreType.REGULAR
        )
        def _(second_barrier):
            for nbr in (left, right):
                pl.semaphore_signal(
                    second_barrier, inc=1,
                    device_id=(nbr,), device_id_type=pl.DeviceIdType.MESH,
                )
            pl.semaphore_wait(second_barrier, 2)

    return pl.pallas_call(
        body,
        out_shape=jax.ShapeDtypeStruct((m_chunk, n_total), jnp.float32),
        in_specs=[
            pl.BlockSpec(memory_space=pltpu.VMEM),
            pl.BlockSpec(memory_space=pltpu.VMEM),
            pl.BlockSpec(memory_space=pltpu.SMEM),
            pl.BlockSpec(memory_space=pltpu.SMEM),
        ],
        out_specs=pl.BlockSpec(memory_space=pl.ANY),
        scratch_shapes=[
            pltpu.VMEM((N_DEV, m_chunk, nt), jnp.int32),
            pltpu.VMEM((N_DEV, m_chunk, nt), jnp.int32),
            pltpu.VMEM((2, m_chunk, nt), jnp.float32),
            pltpu.SemaphoreType.DMA((N_DEV - 1, S)),
            pltpu.SemaphoreType.DMA((N_DEV - 1, S)),
            pltpu.SemaphoreType.DMA((N_DEV - 1, S)),
            pltpu.SemaphoreType.DMA((N_DEV - 1, S)),
            pltpu.SemaphoreType.DMA((2,)),
            pltpu.SemaphoreType.REGULAR,
            pltpu.SemaphoreType.REGULAR,
        ],
        compiler_params=pltpu.CompilerParams(
            collective_id=0,
            vmem_limit_bytes=63 * 1024 * 1024,
        ),
    )(x, w_mat, scale_x, scale_w)


# device time: 601451 ns/iter; 2.0857x vs baseline; 1.0013x over previous
import functools

import jax
import jax.numpy as jnp
from jax import lax
from jax.experimental import pallas as pl
from jax.experimental.pallas import tpu as pltpu

N_DEV = 4
NT = 8
N_ROUNDS = NT // 2
S = 4


def kernel(x, w_mat, scale_x, scale_w):
    m_total, k_shard = x.shape
    k2, n_total = w_mat.shape
    assert k2 == k_shard
    m_chunk = m_total // N_DEV
    nt = n_total // NT
    ms = m_chunk // S

    def body(x_ref, w_ref, sx_ref, sw_ref, out_hbm,
             comm_r, comm_l, stage_ref,
             send_sems_r, recv_sems_r, send_sems_l, recv_sems_l,
             copy_sems, credit_r, credit_l):
        my = lax.axis_index("i")
        left = (my + N_DEV - 1) % N_DEV
        right = (my + 1) % N_DEV

        barrier_sem = pltpu.get_barrier_semaphore()
        for nbr in (left, right):
            pl.semaphore_signal(
                barrier_sem, inc=1,
                device_id=(nbr,), device_id_type=pl.DeviceIdType.MESH,
            )
        pl.semaphore_wait(barrier_sem, 2)

        scale = sx_ref[0] * sw_ref[0]

        def partial(c, t):
            xs = x_ref[pl.ds(c * m_chunk, m_chunk), :]
            ws = w_ref[:, pl.ds(t * nt, nt)]
            return lax.dot_general(
                xs, ws, (((1,), (0,)), ((), ())),
                preferred_element_type=jnp.int32,
            )

        dirs = {
            "r": (comm_r, send_sems_r, recv_sems_r),
            "l": (comm_l, send_sems_l, recv_sems_l),
        }
        pending_send = {}
        pending_copy = {}

        def start_send(tag, h, s, dev):
            comm, ssems, rsems = dirs[tag]
            rdma = pltpu.make_async_remote_copy(
                src_ref=comm.at[h, pl.ds(s * ms, ms), :],
                dst_ref=comm.at[h + 1, pl.ds(s * ms, ms), :],
                send_sem=ssems.at[h, s],
                recv_sem=rsems.at[h, s],
                device_id=(dev,),
                device_id_type=pl.DeviceIdType.MESH,
            )
            key = (tag, h, s)
            if key in pending_send:
                pending_send[key].wait_send()
            rdma.start()
            pending_send[key] = rdma
            return rdma

        for r in range(N_ROUNDS):
            t_r = r
            t_l = N_ROUNDS + r

            comm_r[0, :, :] = partial((my + N_DEV - 1) % N_DEV, t_r)
            comm_l[0, :, :] = partial((my + 1) % N_DEV, t_l)

            if r > 0:
                pl.semaphore_wait(credit_r, 1)
                pl.semaphore_wait(credit_l, 1)

            cur = {}
            for s in range(S):
                cur[("r", 0, s)] = start_send("r", 0, s, right)
                cur[("l", 0, s)] = start_send("l", 0, s, left)

            for h in range(N_DEV - 1):
                last = h == N_DEV - 2
                p_r = partial((my + 2 * N_DEV - 2 - h) % N_DEV, t_r)
                p_l = partial((my + 2 + h) % N_DEV, t_l)

                for s in range(S):
                    rs = pl.ds(s * ms, ms)
                    pv = slice(s * ms, (s + 1) * ms)

                    cur[("r", h, s)].wait_recv()
                    if not last:
                        comm_r[h + 1, rs, :] = comm_r[h + 1, rs, :] + p_r[pv, :]
                        cur[("r", h + 1, s)] = start_send("r", h + 1, s, right)
                    else:
                        if s == 0 and 0 in pending_copy:
                            pending_copy[0].wait()
                        stage_ref[0, rs, :] = (
                            (comm_r[h + 1, rs, :] + p_r[pv, :])
                            .astype(jnp.float32) * scale
                        )

                    cur[("l", h, s)].wait_recv()
                    if not last:
                        comm_l[h + 1, rs, :] = comm_l[h + 1, rs, :] + p_l[pv, :]
                        cur[("l", h + 1, s)] = start_send("l", h + 1, s, left)
                    else:
                        if s == 0 and 1 in pending_copy:
                            pending_copy[1].wait()
                        stage_ref[1, rs, :] = (
                            (comm_l[h + 1, rs, :] + p_l[pv, :])
                            .astype(jnp.float32) * scale
                        )

            for key in list(pending_send):
                pending_send.pop(key).wait_send()
            if r < N_ROUNDS - 1:
                pl.semaphore_signal(
                    credit_r, inc=1,
                    device_id=(left,), device_id_type=pl.DeviceIdType.MESH,
                )
                pl.semaphore_signal(
                    credit_l, inc=1,
                    device_id=(right,), device_id_type=pl.DeviceIdType.MESH,
                )

            for slot, t in ((0, t_r), (1, t_l)):
                copy = pltpu.make_async_copy(
                    stage_ref.at[slot],
                    out_hbm.at[:, pl.ds(t * nt, nt)],
                    copy_sems.at[slot],
                )
                copy.start()
                pending_copy[slot] = copy

        for copy in pending_copy.values():
            copy.wait()

        @functools.partial(
            pl.run_scoped, second_barrier=pltpu.SemaphoreType.REGULAR
        )
        def _(second_barrier):
            for nbr in (left, right):
                pl.semaphore_signal(
                    second_barrier, inc=1,
                    device_id=(nbr,), device_id_type=pl.DeviceIdType.MESH,
                )
            pl.semaphore_wait(second_barrier, 2)

    return pl.pallas_call(
        body,
        out_shape=jax.ShapeDtypeStruct((m_chunk, n_total), jnp.float32),
        in_specs=[
            pl.BlockSpec(memory_space=pltpu.VMEM),
            pl.BlockSpec(memory_space=pltpu.VMEM),
            pl.BlockSpec(memory_space=pltpu.SMEM),
            pl.BlockSpec(memory_space=pltpu.SMEM),
        ],
        out_specs=pl.BlockSpec(memory_space=pl.ANY),
        scratch_shapes=[
            pltpu.VMEM((N_DEV, m_chunk, nt), jnp.int32),
            pltpu.VMEM((N_DEV, m_chunk, nt), jnp.int32),
            pltpu.VMEM((2, m_chunk, nt), jnp.float32),
            pltpu.SemaphoreType.DMA((N_DEV - 1, S)),
            pltpu.SemaphoreType.DMA((N_DEV - 1, S)),
            pltpu.SemaphoreType.DMA((N_DEV - 1, S)),
            pltpu.SemaphoreType.DMA((N_DEV - 1, S)),
            pltpu.SemaphoreType.DMA((2,)),
            pltpu.SemaphoreType.REGULAR,
            pltpu.SemaphoreType.REGULAR,
        ],
        compiler_params=pltpu.CompilerParams(
            collective_id=0,
            vmem_limit_bytes=63 * 1024 * 1024,
        ),
    )(x, w_mat, scale_x, scale_w)


# device time: 269437 ns/iter; 4.6558x vs baseline; 2.2323x over previous
import functools

import jax
import jax.numpy as jnp
from jax import lax
from jax.experimental import pallas as pl
from jax.experimental.pallas import tpu as pltpu

N_DEV = 4
NT = 8


def kernel(x, w_mat, scale_x, scale_w):
    m_total, k_shard = x.shape
    k2, n_total = w_mat.shape
    assert k2 == k_shard
    m_chunk = m_total // N_DEV
    k_total = k_shard * N_DEV
    nh = n_total // 2
    nt = n_total // NT

    def body(x_ref, w_ref, sx_ref, sw_ref, out_hbm,
             wg_ref, xg_ref, stage_ref,
             x_send_sems, x_recv_sems,
             w1_send_sems, w1_recv_sems, w2_send_sems, w2_recv_sems,
             local_sems, copy_sems):
        my = lax.axis_index("i")
        left = (my + N_DEV - 1) % N_DEV
        right = (my + 1) % N_DEV
        opp = (my + 2) % N_DEV

        barrier_sem = pltpu.get_barrier_semaphore()
        for nbr in (left, right, opp):
            pl.semaphore_signal(
                barrier_sem, inc=1,
                device_id=(nbr,), device_id_type=pl.DeviceIdType.MESH,
            )
        pl.semaphore_wait(barrier_sem, 3)

        scale = sx_ref[0] * sw_ref[0]

        def w1_send(dev, sem_slot):
            return pltpu.make_async_remote_copy(
                src_ref=w_ref,
                dst_ref=wg_ref.at[pl.ds(my * k_shard, k_shard), :],
                send_sem=w1_send_sems.at[sem_slot],
                recv_sem=w1_recv_sems.at[sem_slot],
                device_id=(dev,),
                device_id_type=pl.DeviceIdType.MESH,
            )

        def x_send(dev, sem_slot):
            return pltpu.make_async_remote_copy(
                src_ref=x_ref.at[pl.ds(dev * m_chunk, m_chunk), :],
                dst_ref=xg_ref.at[:, pl.ds(my * k_shard, k_shard)],
                send_sem=x_send_sems.at[sem_slot],
                recv_sem=x_recv_sems.at[sem_slot],
                device_id=(dev,),
                device_id_type=pl.DeviceIdType.MESH,
            )

        w1_r = w1_send(right, 0)
        w1_l = w1_send(left, 1)
        x_r = x_send(right, 2)
        x_l = x_send(left, 0)
        x_o = x_send(opp, 1)
        for rdma in (w1_r, w1_l, x_r, x_l, x_o):
            rdma.start()

        w_local = pltpu.make_async_copy(
            w_ref, wg_ref.at[pl.ds(my * k_shard, k_shard), :],
            local_sems.at[0],
        )
        x_local = pltpu.make_async_copy(
            x_ref.at[pl.ds(my * m_chunk, m_chunk), :],
            xg_ref.at[:, pl.ds(my * k_shard, k_shard)],
            local_sems.at[1],
        )
        w_local.start()
        x_local.start()

        w1_r.wait_recv()
        w1_l.wait_recv()

        w2_r = pltpu.make_async_remote_copy(
            src_ref=wg_ref.at[pl.ds(left * k_shard, k_shard), pl.ds(0, nh)],
            dst_ref=wg_ref.at[pl.ds(left * k_shard, k_shard), pl.ds(0, nh)],
            send_sem=w2_send_sems.at[0],
            recv_sem=w2_recv_sems.at[0],
            device_id=(right,),
            device_id_type=pl.DeviceIdType.MESH,
        )
        w2_l = pltpu.make_async_remote_copy(
            src_ref=wg_ref.at[pl.ds(right * k_shard, k_shard), pl.ds(nh, nh)],
            dst_ref=wg_ref.at[pl.ds(right * k_shard, k_shard), pl.ds(nh, nh)],
            send_sem=w2_send_sems.at[1],
            recv_sem=w2_recv_sems.at[1],
            device_id=(left,),
            device_id_type=pl.DeviceIdType.MESH,
        )
        w2_r.start()
        w2_l.start()

        x_r.wait_recv()
        x_l.wait_recv()
        x_o.wait_recv()
        w_local.wait()
        x_local.wait()
        w2_r.wait_recv()
        w2_l.wait_recv()

        pending_copy = {}
        for t in range(NT):
            acc = lax.dot_general(
                xg_ref[:, :], wg_ref[:, pl.ds(t * nt, nt)],
                (((1,), (0,)), ((), ())),
                preferred_element_type=jnp.int32,
            )
            slot = t % 2
            if slot in pending_copy:
                pending_copy[slot].wait()
            stage_ref[slot, :, :] = acc.astype(jnp.float32) * scale
            copy = pltpu.make_async_copy(
                stage_ref.at[slot],
                out_hbm.at[:, pl.ds(t * nt, nt)],
                copy_sems.at[slot],
            )
            copy.start()
            pending_copy[slot] = copy

        for rdma in (w1_r, w1_l, x_r, x_l, x_o, w2_r, w2_l):
            rdma.wait_send()
        for copy in pending_copy.values():
            copy.wait()

        @functools.partial(
            pl.run_scoped, second_barrier=pltpu.SemaphoreType.REGULAR
        )
        def _(second_barrier):
            for nbr in (left, right, opp):
                pl.semaphore_signal(
                    second_barrier, inc=1,
                    device_id=(nbr,), device_id_type=pl.DeviceIdType.MESH,
                )
            pl.semaphore_wait(second_barrier, 3)

    return pl.pallas_call(
        body,
        out_shape=jax.ShapeDtypeStruct((m_chunk, n_total), jnp.float32),
        in_specs=[
            pl.BlockSpec(memory_space=pl.ANY),
            pl.BlockSpec(memory_space=pl.ANY),
            pl.BlockSpec(memory_space=pltpu.SMEM),
            pl.BlockSpec(memory_space=pltpu.SMEM),
        ],
        out_specs=pl.BlockSpec(memory_space=pl.ANY),
        scratch_shapes=[
            pltpu.VMEM((k_total, n_total), jnp.int8),
            pltpu.VMEM((m_chunk, k_total), jnp.int8),
            pltpu.VMEM((2, m_chunk, nt), jnp.float32),
            pltpu.SemaphoreType.DMA((3,)),
            pltpu.SemaphoreType.DMA((3,)),
            pltpu.SemaphoreType.DMA((2,)),
            pltpu.SemaphoreType.DMA((2,)),
            pltpu.SemaphoreType.DMA((2,)),
            pltpu.SemaphoreType.DMA((2,)),
            pltpu.SemaphoreType.DMA((2,)),
            pltpu.SemaphoreType.DMA((2,)),
        ],
        compiler_params=pltpu.CompilerParams(
            collective_id=0,
            vmem_limit_bytes=63 * 1024 * 1024,
        ),
    )(x, w_mat, scale_x, scale_w)


# device time: 202420 ns/iter; 6.1972x vs baseline; 1.3311x over previous
import functools

import jax
import jax.numpy as jnp
from jax import lax
from jax.experimental import pallas as pl
from jax.experimental.pallas import tpu as pltpu

N_DEV = 4
CT = 8
NSLOT = 4


def kernel(x, w_mat, scale_x, scale_w):
    m_total, k_shard = x.shape
    k2, n_total = w_mat.shape
    assert k2 == k_shard
    m_chunk = m_total // N_DEV
    k_total = k_shard * N_DEV
    ct = n_total // CT
    ch = ct // 2

    def body(x_ref, w_ref, sx_ref, sw_ref, out_hbm,
             wg_ref, xg_ref, stage_ref,
             x_send_sems, x_recv_sems,
             w1_send_sems, w1_recv_sems, w2_send_sems, w2_recv_sems,
             wloc_sems, xloc_sem, copy_sems, credit_r, credit_l):
        my = lax.axis_index("i")
        left = (my + N_DEV - 1) % N_DEV
        right = (my + 1) % N_DEV
        opp = (my + 2) % N_DEV

        barrier_sem = pltpu.get_barrier_semaphore()
        for nbr in (left, right, opp):
            pl.semaphore_signal(
                barrier_sem, inc=1,
                device_id=(nbr,), device_id_type=pl.DeviceIdType.MESH,
            )
        pl.semaphore_wait(barrier_sem, 3)

        scale = sx_ref[0] * sw_ref[0]

        def x_send(dev, sem_slot):
            return pltpu.make_async_remote_copy(
                src_ref=x_ref.at[pl.ds(dev * m_chunk, m_chunk), :],
                dst_ref=xg_ref.at[:, pl.ds(my * k_shard, k_shard)],
                send_sem=x_send_sems.at[sem_slot],
                recv_sem=x_recv_sems.at[sem_slot],
                device_id=(dev,),
                device_id_type=pl.DeviceIdType.MESH,
            )

        x_r = x_send(right, 2)
        x_l = x_send(left, 0)
        x_o = x_send(opp, 1)
        x_r.start()
        x_l.start()
        x_o.start()
        x_local = pltpu.make_async_copy(
            x_ref.at[pl.ds(my * m_chunk, m_chunk), :],
            xg_ref.at[:, pl.ds(my * k_shard, k_shard)],
            xloc_sem,
        )
        x_local.start()

        sends_w1 = {}
        sends_w2 = {}
        local_w = {}
        pending_copy = {}

        for it in range(CT + 2):
            t = it
            if t < CT:
                slot = t % NSLOT
                if t >= NSLOT:
                    pl.semaphore_wait(credit_r, 1)
                    pl.semaphore_wait(credit_l, 1)
                    for rd in sends_w1.pop(t - NSLOT):
                        rd.wait_send()
                w1_r = pltpu.make_async_remote_copy(
                    src_ref=w_ref.at[:, pl.ds(t * ct, ct)],
                    dst_ref=wg_ref.at[slot, pl.ds(my * k_shard, k_shard), :],
                    send_sem=w1_send_sems.at[slot, 0],
                    recv_sem=w1_recv_sems.at[slot, 0],
                    device_id=(right,),
                    device_id_type=pl.DeviceIdType.MESH,
                )
                w1_l = pltpu.make_async_remote_copy(
                    src_ref=w_ref.at[:, pl.ds(t * ct, ct)],
                    dst_ref=wg_ref.at[slot, pl.ds(my * k_shard, k_shard), :],
                    send_sem=w1_send_sems.at[slot, 1],
                    recv_sem=w1_recv_sems.at[slot, 1],
                    device_id=(left,),
                    device_id_type=pl.DeviceIdType.MESH,
                )
                w1_r.start()
                w1_l.start()
                sends_w1[t] = (w1_r, w1_l)
                wl = pltpu.make_async_copy(
                    w_ref.at[:, pl.ds(t * ct, ct)],
                    wg_ref.at[slot, pl.ds(my * k_shard, k_shard), :],
                    wloc_sems.at[slot],
                )
                wl.start()
                local_w[t] = wl

            t = it - 1
            if 0 <= t < CT:
                slot = t % NSLOT
                w1_r, w1_l = sends_w1[t]
                w1_r.wait_recv()
                w1_l.wait_recv()
                w2_r = pltpu.make_async_remote_copy(
                    src_ref=wg_ref.at[
                        slot, pl.ds(left * k_shard, k_shard), pl.ds(0, ch)],
                    dst_ref=wg_ref.at[
                        slot, pl.ds(left * k_shard, k_shard), pl.ds(0, ch)],
                    send_sem=w2_send_sems.at[slot, 0],
                    recv_sem=w2_recv_sems.at[slot, 0],
                    device_id=(right,),
                    device_id_type=pl.DeviceIdType.MESH,
                )
                w2_l = pltpu.make_async_remote_copy(
                    src_ref=wg_ref.at[
                        slot, pl.ds(right * k_shard, k_shard), pl.ds(ch, ch)],
                    dst_ref=wg_ref.at[
                        slot, pl.ds(right * k_shard, k_shard), pl.ds(ch, ch)],
                    send_sem=w2_send_sems.at[slot, 1],
                    recv_sem=w2_recv_sems.at[slot, 1],
                    device_id=(left,),
                    device_id_type=pl.DeviceIdType.MESH,
                )
                w2_r.start()
                w2_l.start()
                sends_w2[t] = (w2_r, w2_l)

            t = it - 2
            if 0 <= t < CT:
                slot = t % NSLOT
                w2_r, w2_l = sends_w2[t]
                w2_r.wait_recv()
                w2_l.wait_recv()
                local_w[t].wait()
                if t == 0:
                    x_r.wait_recv()
                    x_l.wait_recv()
                    x_o.wait_recv()
                    x_local.wait()
                acc = lax.dot_general(
                    xg_ref[:, :], wg_ref[slot, :, :],
                    (((1,), (0,)), ((), ())),
                    preferred_element_type=jnp.int32,
                )
                cslot = t % 2
                if cslot in pending_copy:
                    pending_copy[cslot].wait()
                stage_ref[cslot, :, :] = acc.astype(jnp.float32) * scale
                copy = pltpu.make_async_copy(
                    stage_ref.at[cslot],
                    out_hbm.at[:, pl.ds(t * ct, ct)],
                    copy_sems.at[cslot],
                )
                copy.start()
                pending_copy[cslot] = copy
                for rd in sends_w2.pop(t):
                    rd.wait_send()
                if t + NSLOT < CT:
                    pl.semaphore_signal(
                        credit_r, inc=1,
                        device_id=(left,),
                        device_id_type=pl.DeviceIdType.MESH,
                    )
                    pl.semaphore_signal(
                        credit_l, inc=1,
                        device_id=(right,),
                        device_id_type=pl.DeviceIdType.MESH,
                    )

        for pair in sends_w1.values():
            for rd in pair:
                rd.wait_send()
        for rd in (x_r, x_l, x_o):
            rd.wait_send()
        for copy in pending_copy.values():
            copy.wait()

        @functools.partial(
            pl.run_scoped, second_barrier=pltpu.SemaphoreType.REGULAR
        )
        def _(second_barrier):
            for nbr in (left, right, opp):
                pl.semaphore_signal(
                    second_barrier, inc=1,
                    device_id=(nbr,), device_id_type=pl.DeviceIdType.MESH,
                )
            pl.semaphore_wait(second_barrier, 3)

    return pl.pallas_call(
        body,
        out_shape=jax.ShapeDtypeStruct((m_chunk, n_total), jnp.float32),
        in_specs=[
            pl.BlockSpec(memory_space=pl.ANY),
            pl.BlockSpec(memory_space=pl.ANY),
            pl.BlockSpec(memory_space=pltpu.SMEM),
            pl.BlockSpec(memory_space=pltpu.SMEM),
        ],
        out_specs=pl.BlockSpec(memory_space=pl.ANY),
        scratch_shapes=[
            pltpu.VMEM((NSLOT, k_total, n_total // CT), jnp.int8),
            pltpu.VMEM((m_chunk, k_total), jnp.int8),
            pltpu.VMEM((2, m_chunk, n_total // CT), jnp.float32),
            pltpu.SemaphoreType.DMA((3,)),
            pltpu.SemaphoreType.DMA((3,)),
            pltpu.SemaphoreType.DMA((NSLOT, 2)),
            pltpu.SemaphoreType.DMA((NSLOT, 2)),
            pltpu.SemaphoreType.DMA((NSLOT, 2)),
            pltpu.SemaphoreType.DMA((NSLOT, 2)),
            pltpu.SemaphoreType.DMA((NSLOT,)),
            pltpu.SemaphoreType.DMA,
            pltpu.SemaphoreType.DMA((2,)),
            pltpu.SemaphoreType.REGULAR,
            pltpu.SemaphoreType.REGULAR,
        ],
        compiler_params=pltpu.CompilerParams(
            collective_id=0,
            vmem_limit_bytes=63 * 1024 * 1024,
        ),
    )(x, w_mat, scale_x, scale_w)
